# baseline (speedup 1.0000x reference)
import functools

import jax
import jax.numpy as jnp
from jax import lax
from jax.experimental import pallas as pl
from jax.experimental.pallas import tpu as pltpu
from jax.experimental.pallas import tpu_sc as plsc

NS = 16
NC = 2


def _deg_kernel_body(G, NPAD, dst_hbm, ones_hbm, zn_hbm, out_hbm,
                     idx_v, ones_v, zbuf, acc_s, zsem):
    c = lax.axis_index("c")
    s = lax.axis_index("s")
    w = c * NS + s
    pltpu.sync_copy(dst_hbm.at[pl.ds(w * G, G)], idx_v)
    pltpu.sync_copy(ones_hbm, ones_v)
    pltpu.sync_copy(zn_hbm, zbuf)
    R = NPAD // NS
    for t in range(R // 128):
        pltpu.sync_copy(zbuf, acc_s.at[pl.ds(s * R + t * 128, 128)])
    plsc.subcore_barrier()

    for j in range(G):
        pltpu.async_copy(ones_v, acc_s.at[idx_v.at[j]], zsem, add=True)
    for j in range(G):
        pltpu.make_async_copy(ones_v, acc_s.at[idx_v.at[j]], zsem).wait()
    plsc.subcore_barrier()
    for t in range(R // 128):
        pltpu.async_copy(acc_s.at[pl.ds(s * R + t * 128, 128)],
                         out_hbm.at[pl.ds(c * NPAD + s * R + t * 128, 128)],
                         zsem)
    for t in range(R // 128):
        pltpu.make_async_copy(
            acc_s.at[pl.ds(s * R + t * 128, 128)],
            out_hbm.at[pl.ds(c * NPAD + s * R + t * 128, 128)], zsem).wait()


def _agg_kernel_body(G, NPAD, D, h_hbm, src_hbm, dst_hbm, znd_hbm, out_hbm,
                     sidx, dbuf0, dbuf1, bufa, bufb, acc_s,
                     sema, semb, dsem0, dsem1):
    c = lax.axis_index("c")
    s = lax.axis_index("s")
    w = c * NS + s
    bufs = (bufa, bufb)
    sems = (sema, semb)
    dbufs = (dbuf0, dbuf1)
    dsems = (dsem0, dsem1)
    CH = 16
    NCHK = G // CH
    pltpu.sync_copy(src_hbm.at[pl.ds(w * G, G)], sidx)
    for k in range(min(2, NCHK)):
        pltpu.async_copy(dst_hbm.at[pl.ds(w * G + k * CH, CH)], dbufs[k], dsems[k])
    R = NPAD // NS
    pltpu.sync_copy(znd_hbm, bufa)
    for t in range(R // 128):
        pltpu.async_copy(bufa, acc_s.at[pl.ds(s * R + t * 128, 128)], sema)
    for t in range(R // 128):
        pltpu.make_async_copy(bufa, acc_s.at[pl.ds(s * R + t * 128, 128)],
                              sema).wait()
    plsc.subcore_barrier()

    for j in range(min(2, G)):
        pltpu.async_copy(h_hbm.at[sidx.at[j]], bufs[j], sems[j])
    for j in range(G):
        b = j & 1
        k, t = divmod(j, CH)
        if t == 0:
            pltpu.make_async_copy(
                dst_hbm.at[pl.ds(w * G + k * CH, CH)], dbufs[k & 1],
                dsems[k & 1]).wait()
        pltpu.make_async_copy(h_hbm.at[sidx.at[j]], bufs[b], sems[b]).wait()
        pltpu.sync_copy(bufs[b], acc_s.at[dbufs[k & 1].at[t]], add=True)
        if j + 2 < G:
            pltpu.async_copy(h_hbm.at[sidx.at[j + 2]], bufs[b], sems[b])
        if t == CH - 1 and k + 2 < NCHK:
            pltpu.async_copy(dst_hbm.at[pl.ds(w * G + (k + 2) * CH, CH)],
                             dbufs[k & 1], dsems[k & 1])
    plsc.subcore_barrier()
    for t in range(R // 128):
        pltpu.async_copy(acc_s.at[pl.ds(s * R + t * 128, 128)],
                         out_hbm.at[c, pl.ds(s * R + t * 128, 128)], sema)
    for t in range(R // 128):
        pltpu.make_async_copy(acc_s.at[pl.ds(s * R + t * 128, 128)],
                              out_hbm.at[c, pl.ds(s * R + t * 128, 128)],
                              sema).wait()


def _dinv_of(degp_ref):
    deg = degp_ref[0, :] + degp_ref[1, :]
    return jnp.where(deg > 0.0, lax.rsqrt(deg), 0.0)


def _tc0_body(x_ref, w_ref, v_ref, xw_ref, xv_ref):
    xb = x_ref[...]
    xw_ref[...] = jnp.dot(xb, w_ref[...], preferred_element_type=jnp.float32)
    xv_ref[...] = jnp.dot(xb, v_ref[...], preferred_element_type=jnp.float32)


def _tc1s_body(xw_ref, degp_ref, h1s_ref):
    dinv = _dinv_of(degp_ref)
    h1s_ref[...] = xw_ref[...] * dinv[:, None]


def _tc2_body(p_ref, xv1_ref, degp_ref, w2_ref, v2_ref, b1_ref, h2s_ref, xv2_ref):
    dinv = _dinv_of(degp_ref)
    agg = p_ref[0] + p_ref[1]
    pre = agg * dinv[:, None] + xv1_ref[...] + b1_ref[...]
    out1 = jnp.where(pre > 0.0, pre, jnp.exp(pre) - 1.0)
    h2 = jnp.dot(out1, w2_ref[...], preferred_element_type=jnp.float32)
    h2s_ref[...] = h2 * dinv[:, None]
    xv2_ref[...] = jnp.dot(out1, v2_ref[...], preferred_element_type=jnp.float32)


def _tc3_body(p_ref, xv2_ref, degp_ref, b2_ref, out_ref):
    dinv = _dinv_of(degp_ref)
    agg = p_ref[0] + p_ref[1]
    out_ref[...] = agg * dinv[:, None] + xv2_ref[...] + b2_ref[...]


def kernel(x, edge_index, W1, V1, b1, W2, V2, b2):
    N, D = x.shape
    E = edge_index.shape[1]
    NPAD = ((N + 2048) // 2048) * 2048
    G = (E + NC * NS * 128 - 1) // (NC * NS * 128)
    G = ((G + 15) // 16) * 16
    EPAD = G * NC * NS * 128

    pad = (N + jnp.arange(EPAD - E, dtype=jnp.int32) % (NPAD - N)).astype(jnp.int32)
    src2d = jnp.concatenate([edge_index[0], pad]).reshape(EPAD // 128, 128)
    dst2d = jnp.concatenate([edge_index[1], pad]).reshape(EPAD // 128, 128)
    zn = jnp.zeros((128,), jnp.float32)
    znd = jnp.zeros((128, D), jnp.float32)
    ones128 = jnp.ones((128,), jnp.float32)

    mesh = plsc.VectorSubcoreMesh(core_axis_name="c", subcore_axis_name="s",
                                  num_cores=NC, num_subcores=NS)

    deg_call = pl.kernel(
        functools.partial(_deg_kernel_body, G, NPAD),
        out_type=jax.ShapeDtypeStruct((NC * NPAD,), jnp.float32),
        mesh=mesh,
        scratch_types=[
            pltpu.VMEM((G, 128), jnp.int32),
            pltpu.VMEM((128,), jnp.float32),
            pltpu.VMEM((128,), jnp.float32),
            pltpu.VMEM_SHARED((NPAD,), jnp.float32),
            pltpu.SemaphoreType.DMA,
        ],
    )
    degp = deg_call(dst2d, ones128, zn).reshape(NC, NPAD)

    agg_call = pl.kernel(
        functools.partial(_agg_kernel_body, G, NPAD, D),
        out_type=jax.ShapeDtypeStruct((NC, NPAD, D), jnp.float32),
        mesh=mesh,
        scratch_types=[
            pltpu.VMEM((G, 128), jnp.int32),
            pltpu.VMEM((16, 128), jnp.int32),
            pltpu.VMEM((16, 128), jnp.int32),
            pltpu.VMEM((128, D), jnp.float32),
            pltpu.VMEM((128, D), jnp.float32),
            pltpu.VMEM_SHARED((NPAD, D), jnp.float32),
            pltpu.SemaphoreType.DMA,
            pltpu.SemaphoreType.DMA,
            pltpu.SemaphoreType.DMA,
            pltpu.SemaphoreType.DMA,
        ],
    )

    nblk = (N + 127) // 128
    full = pl.BlockSpec((128, D), lambda i: (i, 0))
    wspec = pl.BlockSpec((D, D), lambda i: (0, 0))
    bspec = pl.BlockSpec((1, D), lambda i: (0, 0))
    degspec = pl.BlockSpec((NC, 128), lambda i: (0, i))
    pspec = pl.BlockSpec((NC, 128, D), lambda i: (0, i, 0))

    tc0 = pl.pallas_call(
        _tc0_body,
        grid=(nblk,),
        in_specs=[full, wspec, wspec],
        out_specs=[full, full],
        out_shape=[jax.ShapeDtypeStruct((NPAD, D), jnp.float32)] * 2,
    )
    xw1, xv1 = tc0(x, W1, V1)

    tc1s = pl.pallas_call(
        _tc1s_body,
        grid=(nblk,),
        in_specs=[full, degspec],
        out_specs=full,
        out_shape=jax.ShapeDtypeStruct((NPAD, D), jnp.float32),
    )
    h1s = tc1s(xw1, degp)

    p1 = agg_call(h1s, src2d, dst2d, znd)

    tc2 = pl.pallas_call(
        _tc2_body,
        grid=(nblk,),
        in_specs=[pspec, full, degspec, wspec, wspec, bspec],
        out_specs=[full, full],
        out_shape=[jax.ShapeDtypeStruct((NPAD, D), jnp.float32)] * 2,
    )
    h2s, xv2 = tc2(p1, xv1, degp, W2, V2, b1.reshape(1, D))

    p2 = agg_call(h2s, src2d, dst2d, znd)

    tc3 = pl.pallas_call(
        _tc3_body,
        grid=(nblk,),
        in_specs=[pspec, full, degspec, bspec],
        out_specs=full,
        out_shape=jax.ShapeDtypeStruct((N, D), jnp.float32),
    )
    return tc3(p2, xv2, degp, b2.reshape(1, D))


# speedup vs baseline: 1.0786x; 1.0786x over previous
import functools

import jax
import jax.numpy as jnp
from jax import lax
from jax.experimental import pallas as pl
from jax.experimental.pallas import tpu as pltpu
from jax.experimental.pallas import tpu_sc as plsc

NS = 16
NC = 2


def _deg_kernel_body(G, NPAD, dst_hbm, ones_hbm, zn_hbm, out_hbm,
                     idx_v, ones_v, zbuf, acc_s, zsem):
    c = lax.axis_index("c")
    s = lax.axis_index("s")
    w = c * NS + s
    pltpu.sync_copy(dst_hbm.at[pl.ds(w * G, G)], idx_v)
    pltpu.sync_copy(ones_hbm, ones_v)
    pltpu.sync_copy(zn_hbm, zbuf)
    R = NPAD // NS
    for t in range(R // 128):
        pltpu.sync_copy(zbuf, acc_s.at[pl.ds(s * R + t * 128, 128)])
    plsc.subcore_barrier()

    for j in range(G):
        pltpu.async_copy(ones_v, acc_s.at[idx_v.at[j]], zsem, add=True)
    for j in range(G):
        pltpu.make_async_copy(ones_v, acc_s.at[idx_v.at[j]], zsem).wait()
    plsc.subcore_barrier()
    for t in range(R // 128):
        pltpu.async_copy(acc_s.at[pl.ds(s * R + t * 128, 128)],
                         out_hbm.at[pl.ds(c * NPAD + s * R + t * 128, 128)],
                         zsem)
    for t in range(R // 128):
        pltpu.make_async_copy(
            acc_s.at[pl.ds(s * R + t * 128, 128)],
            out_hbm.at[pl.ds(c * NPAD + s * R + t * 128, 128)], zsem).wait()


def _agg_kernel_body(G, NPAD, D, h_hbm, src_hbm, dst_hbm, znd_hbm, out_hbm,
                     sidx, dbuf0, dbuf1, bufa, bufb, acc_s,
                     sema, semb, dsem0, dsem1):
    c = lax.axis_index("c")
    s = lax.axis_index("s")
    w = c * NS + s
    bufs = (bufa, bufb)
    sems = (sema, semb)
    dbufs = (dbuf0, dbuf1)
    dsems = (dsem0, dsem1)
    CH = 16
    NCHK = G // CH
    pltpu.sync_copy(src_hbm.at[pl.ds(w * G, G)], sidx)
    for k in range(min(2, NCHK)):
        pltpu.async_copy(dst_hbm.at[pl.ds(w * G + k * CH, CH)], dbufs[k], dsems[k])
    R = NPAD // NS
    pltpu.sync_copy(znd_hbm, bufa)
    for t in range(R // 128):
        pltpu.async_copy(bufa, acc_s.at[pl.ds(s * R + t * 128, 128)], sema)
    for t in range(R // 128):
        pltpu.make_async_copy(bufa, acc_s.at[pl.ds(s * R + t * 128, 128)],
                              sema).wait()
    plsc.subcore_barrier()

    for j in range(min(2, G)):
        pltpu.async_copy(h_hbm.at[sidx.at[j]], bufs[j], sems[j])
    for j in range(G):
        b = j & 1
        k, t = divmod(j, CH)
        if t == 0:
            pltpu.make_async_copy(
                dst_hbm.at[pl.ds(w * G + k * CH, CH)], dbufs[k & 1],
                dsems[k & 1]).wait()
        pltpu.make_async_copy(h_hbm.at[sidx.at[j]], bufs[b], sems[b]).wait()
        pltpu.sync_copy(bufs[b], acc_s.at[dbufs[k & 1].at[t]], add=True)
        if j + 2 < G:
            pltpu.async_copy(h_hbm.at[sidx.at[j + 2]], bufs[b], sems[b])
        if t == CH - 1 and k + 2 < NCHK:
            pltpu.async_copy(dst_hbm.at[pl.ds(w * G + (k + 2) * CH, CH)],
                             dbufs[k & 1], dsems[k & 1])
    plsc.subcore_barrier()
    for t in range(R // 128):
        pltpu.async_copy(acc_s.at[pl.ds(s * R + t * 128, 128)],
                         out_hbm.at[c, pl.ds(s * R + t * 128, 128)], sema)
    for t in range(R // 128):
        pltpu.make_async_copy(acc_s.at[pl.ds(s * R + t * 128, 128)],
                              out_hbm.at[c, pl.ds(s * R + t * 128, 128)],
                              sema).wait()


def _dinv_of(degp_ref):
    deg = degp_ref[0, :] + degp_ref[1, :]
    return jnp.where(deg > 0.0, lax.rsqrt(deg), 0.0)


def _tc1_body(x_ref, w_ref, v_ref, degp_ref, h1s_ref, xv1_ref):
    dinv = _dinv_of(degp_ref)
    xb = x_ref[...]
    h = jnp.dot(xb, w_ref[...], preferred_element_type=jnp.float32)
    h1s_ref[...] = h * dinv[:, None]
    xv1_ref[...] = jnp.dot(xb, v_ref[...], preferred_element_type=jnp.float32)


def _tc2_body(p_ref, xv1_ref, degp_ref, w2_ref, v2_ref, b1_ref, h2s_ref, xv2_ref):
    dinv = _dinv_of(degp_ref)
    agg = p_ref[0] + p_ref[1]
    pre = agg * dinv[:, None] + xv1_ref[...] + b1_ref[...]
    out1 = jnp.where(pre > 0.0, pre, jnp.exp(pre) - 1.0)
    h2 = jnp.dot(out1, w2_ref[...], preferred_element_type=jnp.float32)
    h2s_ref[...] = h2 * dinv[:, None]
    xv2_ref[...] = jnp.dot(out1, v2_ref[...], preferred_element_type=jnp.float32)


def _tc3_body(p_ref, xv2_ref, degp_ref, b2_ref, out_ref):
    dinv = _dinv_of(degp_ref)
    agg = p_ref[0] + p_ref[1]
    out_ref[...] = agg * dinv[:, None] + xv2_ref[...] + b2_ref[...]


def kernel(x, edge_index, W1, V1, b1, W2, V2, b2):
    N, D = x.shape
    E = edge_index.shape[1]
    NPAD = ((N + 2048) // 2048) * 2048
    G = (E + NC * NS * 128 - 1) // (NC * NS * 128)
    G = ((G + 15) // 16) * 16
    EPAD = G * NC * NS * 128

    pad = (N + jnp.arange(EPAD - E, dtype=jnp.int32) % (NPAD - N)).astype(jnp.int32)
    src2d = jnp.concatenate([edge_index[0], pad]).reshape(EPAD // 128, 128)
    dst2d = jnp.concatenate([edge_index[1], pad]).reshape(EPAD // 128, 128)
    zn = jnp.zeros((128,), jnp.float32)
    znd = jnp.zeros((128, D), jnp.float32)
    ones128 = jnp.ones((128,), jnp.float32)

    mesh = plsc.VectorSubcoreMesh(core_axis_name="c", subcore_axis_name="s",
                                  num_cores=NC, num_subcores=NS)

    deg_call = pl.kernel(
        functools.partial(_deg_kernel_body, G, NPAD),
        out_type=jax.ShapeDtypeStruct((NC * NPAD,), jnp.float32),
        mesh=mesh,
        scratch_types=[
            pltpu.VMEM((G, 128), jnp.int32),
            pltpu.VMEM((128,), jnp.float32),
            pltpu.VMEM((128,), jnp.float32),
            pltpu.VMEM_SHARED((NPAD,), jnp.float32),
            pltpu.SemaphoreType.DMA,
        ],
    )
    degp = deg_call(dst2d, ones128, zn).reshape(NC, NPAD)

    agg_call = pl.kernel(
        functools.partial(_agg_kernel_body, G, NPAD, D),
        out_type=jax.ShapeDtypeStruct((NC, NPAD, D), jnp.float32),
        mesh=mesh,
        scratch_types=[
            pltpu.VMEM((G, 128), jnp.int32),
            pltpu.VMEM((16, 128), jnp.int32),
            pltpu.VMEM((16, 128), jnp.int32),
            pltpu.VMEM((128, D), jnp.float32),
            pltpu.VMEM((128, D), jnp.float32),
            pltpu.VMEM_SHARED((NPAD, D), jnp.float32),
            pltpu.SemaphoreType.DMA,
            pltpu.SemaphoreType.DMA,
            pltpu.SemaphoreType.DMA,
            pltpu.SemaphoreType.DMA,
        ],
    )

    nblk = (N + 127) // 128
    full = pl.BlockSpec((128, D), lambda i: (i, 0))
    wspec = pl.BlockSpec((D, D), lambda i: (0, 0))
    bspec = pl.BlockSpec((1, D), lambda i: (0, 0))
    degspec = pl.BlockSpec((NC, 128), lambda i: (0, i))
    pspec = pl.BlockSpec((NC, 128, D), lambda i: (0, i, 0))

    tc1 = pl.pallas_call(
        _tc1_body,
        grid=(nblk,),
        in_specs=[full, wspec, wspec, degspec],
        out_specs=[full, full],
        out_shape=[jax.ShapeDtypeStruct((NPAD, D), jnp.float32)] * 2,
    )
    h1s, xv1 = tc1(x, W1, V1, degp)

    p1 = agg_call(h1s, src2d, dst2d, znd)

    tc2 = pl.pallas_call(
        _tc2_body,
        grid=(nblk,),
        in_specs=[pspec, full, degspec, wspec, wspec, bspec],
        out_specs=[full, full],
        out_shape=[jax.ShapeDtypeStruct((NPAD, D), jnp.float32)] * 2,
    )
    h2s, xv2 = tc2(p1, xv1, degp, W2, V2, b1.reshape(1, D))

    p2 = agg_call(h2s, src2d, dst2d, znd)

    tc3 = pl.pallas_call(
        _tc3_body,
        grid=(nblk,),
        in_specs=[pspec, full, degspec, bspec],
        out_specs=full,
        out_shape=jax.ShapeDtypeStruct((N, D), jnp.float32),
    )
    return tc3(p2, xv2, degp, b2.reshape(1, D))
